# Initial kernel scaffold; baseline (speedup 1.0000x reference)
import functools

import jax
import jax.numpy as jnp
from jax import lax
from jax.experimental import pallas as pl
from jax.experimental.pallas import tpu as pltpu
from jax.experimental.pallas import tpu_sc as plsc

_B, _S, _H = 256, 512, 768
_EPS = 1e-12
_W = 64


def _sc_gather(word_table, ids):
    n = ids.shape[1]
    mesh = plsc.VectorSubcoreMesh(core_axis_name="c", subcore_axis_name="s")

    @functools.partial(
        pl.kernel,
        out_type=jax.ShapeDtypeStruct((n, _H), jnp.float32),
        mesh=mesh,
    )
    def gather_kernel(table_hbm, idx_hbm, out_hbm):
        def body(idx_vmem, out_vmem):
            pltpu.sync_copy(table_hbm.at[idx_vmem.at[0]], out_vmem)

        pltpu.emit_pipeline(
            body,
            grid=(n // _W,),
            in_specs=[pl.BlockSpec((1, _W), index_map=lambda i: (0, i))],
            out_specs=[pl.BlockSpec((_W, _H), index_map=lambda i: (i, 0))],
            core_axis_name=("c", "s"),
            dimension_semantics=(pltpu.PARALLEL,),
        )(idx_hbm, out_hbm)

    return gather_kernel(word_table, ids)


def _ln_body(g_ref, tt_ref, pos2_ref, dt_ref, gamma_ref, beta_ref, o_ref):
    ttf = tt_ref[...]
    x = g_ref[0] + pos2_ref[...] + ttf * dt_ref[...]
    mean = jnp.mean(x, axis=-1, keepdims=True)
    xc = x - mean
    var = jnp.mean(xc * xc, axis=-1, keepdims=True)
    o_ref[0] = xc * lax.rsqrt(var + _EPS) * gamma_ref[...] + beta_ref[...]


def kernel(input_ids, token_type_ids, word_table, pos_table, type_table,
           ln_gamma, ln_beta):
    ids = input_ids.reshape(1, _B * _S).astype(jnp.int32)
    gathered = _sc_gather(word_table, ids).reshape(_B, _S, _H)

    pos2 = pos_table + type_table[0]
    dt = (type_table[1] - type_table[0]).reshape(1, _H)
    tt_t = token_type_ids.astype(jnp.float32).T
    gamma = ln_gamma.reshape(1, _H)
    beta = ln_beta.reshape(1, _H)

    return pl.pallas_call(
        _ln_body,
        grid=(_B,),
        in_specs=[
            pl.BlockSpec((1, _S, _H), lambda b: (b, 0, 0)),
            pl.BlockSpec((_S, 1), lambda b: (0, b)),
            pl.BlockSpec((_S, _H), lambda b: (0, 0)),
            pl.BlockSpec((1, _H), lambda b: (0, 0)),
            pl.BlockSpec((1, _H), lambda b: (0, 0)),
            pl.BlockSpec((1, _H), lambda b: (0, 0)),
        ],
        out_specs=pl.BlockSpec((1, _S, _H), lambda b: (b, 0, 0)),
        out_shape=jax.ShapeDtypeStruct((_B, _S, _H), jnp.float32),
    )(gathered, tt_t, pos2, dt, gamma, beta)


# speedup vs baseline: 2.1005x; 2.1005x over previous
import functools

import jax
import jax.numpy as jnp
from jax import lax
from jax.experimental import pallas as pl
from jax.experimental.pallas import tpu as pltpu
from jax.experimental.pallas import tpu_sc as plsc

_B, _S, _H = 256, 512, 768
_EPS = 1e-12
_W = 64


_NC, _NS = 2, 16
_NW = _NC * _NS


def _sc_gather(word_table, ids):
    n = ids.shape[0]
    per_w = n // _NW
    ch = _W
    n_ch = per_w // ch
    mesh = plsc.VectorSubcoreMesh(core_axis_name="c", subcore_axis_name="s")

    @functools.partial(
        pl.kernel,
        out_type=jax.ShapeDtypeStruct((n, _H), jnp.float32),
        mesh=mesh,
        scratch_types=[
            pltpu.VMEM((per_w,), jnp.int32),
            pltpu.VMEM((ch, _H), jnp.float32),
            pltpu.VMEM((ch, _H), jnp.float32),
            pltpu.SemaphoreType.DMA,
            pltpu.SemaphoreType.DMA,
        ],
    )
    def gather_kernel(table_hbm, idx_hbm, out_hbm, idx_v, buf0, buf1,
                      sem0, sem1):
        wid = lax.axis_index("s") * _NC + lax.axis_index("c")
        base = wid * per_w
        pltpu.sync_copy(idx_hbm.at[pl.ds(base, per_w)], idx_v)

        def gather(c, buf, sem):
            pltpu.async_copy(table_hbm.at[idx_v.at[pl.ds(c * ch, ch)]],
                             buf, sem)

        def wait_store(c, buf, sem):
            pltpu.make_async_copy(table_hbm.at[idx_v.at[pl.ds(c * ch, ch)]],
                                  buf, sem).wait()
            pltpu.sync_copy(buf, out_hbm.at[pl.ds(base + c * ch, ch)])

        gather(0, buf0, sem0)

        @pl.loop(0, n_ch, step=2)
        def _(c):
            gather(c + 1, buf1, sem1)
            wait_store(c, buf0, sem0)

            @pl.when(c + 2 < n_ch)
            def _():
                gather(c + 2, buf0, sem0)

            wait_store(c + 1, buf1, sem1)

    return gather_kernel(word_table, ids)


def _ln_body(g_ref, tt_ref, pos2_ref, dt_ref, gamma_ref, beta_ref, o_ref):
    ttf = tt_ref[0]
    type_add = lax.dot_general(ttf, dt_ref[...], (((0,), (0,)), ((), ())),
                               preferred_element_type=jnp.float32)
    x = g_ref[0] + pos2_ref[...] + type_add
    mean = jnp.mean(x, axis=-1, keepdims=True)
    xc = x - mean
    var = jnp.mean(xc * xc, axis=-1, keepdims=True)
    o_ref[0] = xc * lax.rsqrt(var + _EPS) * gamma_ref[...] + beta_ref[...]


def kernel(input_ids, token_type_ids, word_table, pos_table, type_table,
           ln_gamma, ln_beta):
    ids = input_ids.reshape(_B * _S).astype(jnp.int32)
    gathered = _sc_gather(word_table, ids).reshape(_B, _S, _H)

    pos2 = pos_table + type_table[0]
    dt = (type_table[1] - type_table[0]).reshape(1, _H)
    tt3 = token_type_ids.astype(jnp.float32).reshape(_B, 1, _S)
    gamma = ln_gamma.reshape(1, _H)
    beta = ln_beta.reshape(1, _H)

    return pl.pallas_call(
        _ln_body,
        grid=(_B,),
        in_specs=[
            pl.BlockSpec((1, _S, _H), lambda b: (b, 0, 0)),
            pl.BlockSpec((1, 1, _S), lambda b: (b, 0, 0)),
            pl.BlockSpec((_S, _H), lambda b: (0, 0)),
            pl.BlockSpec((1, _H), lambda b: (0, 0)),
            pl.BlockSpec((1, _H), lambda b: (0, 0)),
            pl.BlockSpec((1, _H), lambda b: (0, 0)),
        ],
        out_specs=pl.BlockSpec((1, _S, _H), lambda b: (b, 0, 0)),
        out_shape=jax.ShapeDtypeStruct((_B, _S, _H), jnp.float32),
    )(gathered, tt3, pos2, dt, gamma, beta)


# speedup vs baseline: 2.3168x; 1.1029x over previous
import functools

import jax
import jax.numpy as jnp
from jax import lax
from jax.experimental import pallas as pl
from jax.experimental.pallas import tpu as pltpu
from jax.experimental.pallas import tpu_sc as plsc

_B, _S, _H = 256, 512, 768
_EPS = 1e-12
_W = 64


_NC, _NS = 2, 16
_NW = _NC * _NS


def _sc_gather(word_table, ids):
    n = ids.shape[0]
    per_w = n // _NW
    ch = _W
    n_ch = per_w // ch
    mesh = plsc.VectorSubcoreMesh(core_axis_name="c", subcore_axis_name="s")

    @functools.partial(
        pl.kernel,
        out_type=jax.ShapeDtypeStruct((n, _H), jnp.float32),
        mesh=mesh,
        scratch_types=[
            pltpu.VMEM((per_w,), jnp.int32),
            pltpu.VMEM((ch, _H), jnp.float32),
            pltpu.VMEM((ch, _H), jnp.float32),
            pltpu.SemaphoreType.DMA,
            pltpu.SemaphoreType.DMA,
        ],
    )
    def gather_kernel(table_hbm, idx_hbm, out_hbm, idx_v, buf0, buf1,
                      sem0, sem1):
        wid = lax.axis_index("s") * _NC + lax.axis_index("c")
        base = wid * per_w
        pltpu.sync_copy(idx_hbm.at[pl.ds(base, per_w)], idx_v)

        def gather(c, buf, sem):
            pltpu.async_copy(table_hbm.at[idx_v.at[pl.ds(c * ch, ch)]],
                             buf, sem)

        def wait_store(c, buf, sem):
            pltpu.make_async_copy(table_hbm.at[idx_v.at[pl.ds(c * ch, ch)]],
                                  buf, sem).wait()
            pltpu.sync_copy(buf, out_hbm.at[pl.ds(base + c * ch, ch)])

        gather(0, buf0, sem0)

        @pl.loop(0, n_ch, step=2)
        def _(c):
            gather(c + 1, buf1, sem1)
            wait_store(c, buf0, sem0)

            @pl.when(c + 2 < n_ch)
            def _():
                gather(c + 2, buf0, sem0)

            wait_store(c + 1, buf1, sem1)

    return gather_kernel(word_table, ids)


def _ln_compute(g_ref, tt_ref, pos2_ref, dt_ref, gamma_ref, beta_ref, o_ref,
                out_row):
    ttf = tt_ref[0]
    type_add = lax.dot_general(ttf, dt_ref[...], (((0,), (0,)), ((), ())),
                               preferred_element_type=jnp.float32)
    x = g_ref[0] + pos2_ref[...] + type_add
    mean = jnp.mean(x, axis=-1, keepdims=True)
    xc = x - mean
    var = jnp.mean(xc * xc, axis=-1, keepdims=True)
    o_ref[out_row] = xc * lax.rsqrt(var + _EPS) * gamma_ref[...] + beta_ref[...]


def _ln_body_first(g_ref, tt_ref, pos2_ref, dt_ref, gamma_ref, beta_ref,
                   o_ref):
    _ln_compute(g_ref, tt_ref, pos2_ref, dt_ref, gamma_ref, beta_ref, o_ref, 0)


def _ln_body_chained(carry_ref, g_ref, tt_ref, pos2_ref, dt_ref, gamma_ref,
                     beta_ref, o_ref):
    del carry_ref
    _ln_compute(g_ref, tt_ref, pos2_ref, dt_ref, gamma_ref, beta_ref, o_ref, 0)


_K = 4


def kernel(input_ids, token_type_ids, word_table, pos_table, type_table,
           ln_gamma, ln_beta):
    ids = input_ids.reshape(_B * _S).astype(jnp.int32)

    pos2 = pos_table + type_table[0]
    dt = (type_table[1] - type_table[0]).reshape(1, _H)
    tt3 = token_type_ids.astype(jnp.float32).reshape(_B, 1, _S)
    gamma = ln_gamma.reshape(1, _H)
    beta = ln_beta.reshape(1, _H)

    bk = _B // _K
    nk = bk * _S
    gs = [
        _sc_gather(word_table, ids[k * nk:(k + 1) * nk]).reshape(bk, _S, _H)
        for k in range(_K)
    ]

    const_specs = [
        pl.BlockSpec((_S, _H), lambda b: (0, 0)),
        pl.BlockSpec((1, _H), lambda b: (0, 0)),
        pl.BlockSpec((1, _H), lambda b: (0, 0)),
        pl.BlockSpec((1, _H), lambda b: (0, 0)),
    ]
    out_shape = jax.ShapeDtypeStruct((_B, _S, _H), jnp.float32)

    out = pl.pallas_call(
        _ln_body_first,
        grid=(bk,),
        in_specs=[
            pl.BlockSpec((1, _S, _H), lambda b: (b, 0, 0)),
            pl.BlockSpec((1, 1, _S), lambda b: (b, 0, 0)),
            *const_specs,
        ],
        out_specs=pl.BlockSpec((1, _S, _H), lambda b: (b, 0, 0)),
        out_shape=out_shape,
    )(gs[0], tt3[0:bk], pos2, dt, gamma, beta)

    for k in range(1, _K):
        out = pl.pallas_call(
            _ln_body_chained,
            grid=(bk,),
            in_specs=[
                pl.BlockSpec(memory_space=pl.ANY),
                pl.BlockSpec((1, _S, _H), lambda b: (b, 0, 0)),
                pl.BlockSpec((1, 1, _S), lambda b: (b, 0, 0)),
                *const_specs,
            ],
            out_specs=pl.BlockSpec(
                (1, _S, _H), lambda b, _k=k: (b + _k * bk, 0, 0)),
            out_shape=out_shape,
            input_output_aliases={0: 0},
        )(out, gs[k], tt3[k * bk:(k + 1) * bk], pos2, dt, gamma, beta)

    return out
